# Initial kernel scaffold; baseline (speedup 1.0000x reference)
import functools

import jax
import jax.numpy as jnp
from jax import lax
from jax.experimental import pallas as pl
from jax.experimental.pallas import tpu as pltpu
from jax.experimental.pallas import tpu_sc as plsc

DIM = 64
NC = 2
NS = 16
NW = NC * NS

CH = 128


@functools.lru_cache(maxsize=None)
def _make_gather(n: int, vocab: int):
    per_w = n // NW
    n_ch = per_w // CH
    mesh = plsc.VectorSubcoreMesh(core_axis_name="c", subcore_axis_name="s")

    @functools.partial(
        pl.kernel,
        mesh=mesh,
        out_type=jax.ShapeDtypeStruct((n, DIM), jnp.float32),
        scratch_types=[
            pltpu.VMEM((per_w,), jnp.int32),
            pltpu.VMEM((CH, DIM), jnp.float32),
            pltpu.SemaphoreType.DMA,
        ],
    )
    def gather_kernel(idx_hbm, table_hbm, out_hbm, idx_v, rows_v, sem):
        wid = lax.axis_index("s") * NC + lax.axis_index("c")
        base = wid * per_w
        pltpu.sync_copy(idx_hbm.at[pl.ds(base, per_w)], idx_v)

        def body(i, carry):
            off = i * CH
            pltpu.async_copy(
                table_hbm.at[idx_v.at[pl.ds(off, CH)]], rows_v, sem
            ).wait()
            pltpu.sync_copy(rows_v, out_hbm.at[pl.ds(base + off, CH)])
            return carry

        lax.fori_loop(0, n_ch, body, 0)

    return gather_kernel


def kernel(x, W):
    b, l = x.shape
    idx = x.reshape(-1)
    out = _make_gather(idx.shape[0], W.shape[0])(idx, W)
    return out.reshape(b, l, DIM)


# speedup vs baseline: 3.5371x; 3.5371x over previous
import functools

import jax
import jax.numpy as jnp
from jax import lax
from jax.experimental import pallas as pl
from jax.experimental.pallas import tpu as pltpu
from jax.experimental.pallas import tpu_sc as plsc

DIM = 64
NC = 2
NS = 16
NW = NC * NS

CH = 128


@functools.lru_cache(maxsize=None)
def _make_gather(n: int, vocab: int):
    per_w = n // NW
    n_ch = per_w // CH
    mesh = plsc.VectorSubcoreMesh(core_axis_name="c", subcore_axis_name="s")

    @functools.partial(
        pl.kernel,
        mesh=mesh,
        out_type=jax.ShapeDtypeStruct((n, DIM), jnp.float32),
        scratch_types=[
            pltpu.VMEM((per_w,), jnp.int32),
            pltpu.VMEM((CH, DIM), jnp.float32),
            pltpu.SemaphoreType.DMA,
        ],
        compiler_params=pltpu.CompilerParams(use_tc_tiling_on_sc=False),
    )
    def gather_kernel(idx_hbm, table_hbm, out_hbm, idx_v, rows_v, sem):
        wid = lax.axis_index("s") * NC + lax.axis_index("c")
        base = wid * per_w
        pltpu.sync_copy(idx_hbm.at[pl.ds(base, per_w)], idx_v)

        def body(i, carry):
            off = i * CH
            pltpu.async_copy(
                table_hbm.at[idx_v.at[pl.ds(off, CH)]], rows_v, sem
            ).wait()
            pltpu.sync_copy(rows_v, out_hbm.at[pl.ds(base + off, CH)])
            return carry

        lax.fori_loop(0, n_ch, body, 0)

    return gather_kernel


def kernel(x, W):
    b, l = x.shape
    idx = x.reshape(-1)
    out = _make_gather(idx.shape[0], W.shape[0])(idx, W)
    return out.reshape(b, l, DIM)


# speedup vs baseline: 4.2135x; 1.1913x over previous
import functools

import jax
import jax.numpy as jnp
from jax import lax
from jax.experimental import pallas as pl
from jax.experimental.pallas import tpu as pltpu
from jax.experimental.pallas import tpu_sc as plsc

DIM = 64
NC = 2
NS = 16
NW = NC * NS

CH = 128
K = 4


@functools.lru_cache(maxsize=None)
def _make_gather(n: int, vocab: int):
    per_w = n // NW
    n_ch = per_w // CH
    n_grp = n_ch // K
    assert n == per_w * NW and per_w == n_ch * CH and n_ch == n_grp * K
    assert n_grp % 2 == 0
    mesh = plsc.VectorSubcoreMesh(core_axis_name="c", subcore_axis_name="s")

    @functools.partial(
        pl.kernel,
        mesh=mesh,
        out_type=jax.ShapeDtypeStruct((n, DIM), jnp.float32),
        scratch_types=[
            pltpu.VMEM((per_w,), jnp.int32),
            pltpu.VMEM((2, K, CH, DIM), jnp.float32),
            pltpu.SemaphoreType.DMA,
            pltpu.SemaphoreType.DMA,
            pltpu.SemaphoreType.DMA,
            pltpu.SemaphoreType.DMA,
        ],
        compiler_params=pltpu.CompilerParams(use_tc_tiling_on_sc=False),
    )
    def gather_kernel(idx_hbm, table_hbm, out_hbm, idx_v, rows_v,
                      gsem0, gsem1, ssem0, ssem1):
        gsem = (gsem0, gsem1)
        ssem = (ssem0, ssem1)
        wid = lax.axis_index("s") * NC + lax.axis_index("c")
        base = wid * per_w
        pltpu.sync_copy(idx_hbm.at[pl.ds(base, per_w)], idx_v)

        def fire_gathers(r, g):
            for j in range(K):
                off = (g * K + j) * CH
                pltpu.async_copy(
                    table_hbm.at[idx_v.at[pl.ds(off, CH)]],
                    rows_v.at[r, j], gsem[r])

        def drain_gathers(r):
            for j in range(K):
                pltpu.make_async_copy(
                    table_hbm.at[pl.ds(0, CH)], rows_v.at[r, j],
                    gsem[r]).wait()

        def fire_scatters(r, g):
            for j in range(K):
                off = (g * K + j) * CH
                pltpu.async_copy(
                    rows_v.at[r, j], out_hbm.at[pl.ds(base + off, CH)],
                    ssem[r])

        def drain_scatters(r):
            for j in range(K):
                pltpu.make_async_copy(
                    rows_v.at[r, j], out_hbm.at[pl.ds(base, CH)],
                    ssem[r]).wait()

        fire_gathers(0, 0)
        fire_gathers(1, 1)

        @pl.loop(0, n_grp, step=2)
        def _(g):
            drain_gathers(0)
            fire_scatters(0, g)
            drain_gathers(1)
            fire_scatters(1, g + 1)
            drain_scatters(0)

            @pl.when(g + 2 < n_grp)
            def _():
                fire_gathers(0, g + 2)

            drain_scatters(1)

            @pl.when(g + 3 < n_grp)
            def _():
                fire_gathers(1, g + 3)

    return gather_kernel


def kernel(x, W):
    b, l = x.shape
    idx = x.reshape(-1)
    out = _make_gather(idx.shape[0], W.shape[0])(idx, W)
    return out.reshape(b, l, DIM)


# speedup vs baseline: 4.2139x; 1.0001x over previous
import functools

import jax
import jax.numpy as jnp
from jax import lax
from jax.experimental import pallas as pl
from jax.experimental.pallas import tpu as pltpu
from jax.experimental.pallas import tpu_sc as plsc

DIM = 64
NC = 2
NS = 16
NW = NC * NS

CH = 128
K = 5
KCH = K * CH


@functools.lru_cache(maxsize=None)
def _make_gather(n: int, vocab: int):
    per_w = n // NW
    n_grp = per_w // KCH
    assert n == per_w * NW and per_w == n_grp * KCH
    assert n_grp % 2 == 0
    mesh = plsc.VectorSubcoreMesh(core_axis_name="c", subcore_axis_name="s")

    @functools.partial(
        pl.kernel,
        mesh=mesh,
        out_type=jax.ShapeDtypeStruct((n, DIM), jnp.float32),
        scratch_types=[
            pltpu.VMEM((per_w,), jnp.int32),
            pltpu.VMEM((2, KCH, DIM), jnp.float32),
            pltpu.SemaphoreType.DMA,
            pltpu.SemaphoreType.DMA,
            pltpu.SemaphoreType.DMA,
            pltpu.SemaphoreType.DMA,
        ],
        compiler_params=pltpu.CompilerParams(use_tc_tiling_on_sc=False),
    )
    def gather_kernel(idx_hbm, table_hbm, out_hbm, idx_v, rows_v,
                      gsem0, gsem1, ssem0, ssem1):
        gsem = (gsem0, gsem1)
        ssem = (ssem0, ssem1)
        wid = lax.axis_index("s") * NC + lax.axis_index("c")
        base = wid * per_w
        pltpu.sync_copy(idx_hbm.at[pl.ds(base, per_w)], idx_v)

        def fire_gathers(r, g):
            for j in range(K):
                off = g * KCH + j * CH
                pltpu.async_copy(
                    table_hbm.at[idx_v.at[pl.ds(off, CH)]],
                    rows_v.at[r, pl.ds(j * CH, CH)], gsem[r])

        def drain_gathers(r):
            pltpu.make_async_copy(
                table_hbm.at[pl.ds(0, KCH)], rows_v.at[r], gsem[r]).wait()

        def fire_scatter(r, g):
            pltpu.async_copy(
                rows_v.at[r], out_hbm.at[pl.ds(base + g * KCH, KCH)],
                ssem[r])

        def drain_scatter(r):
            pltpu.make_async_copy(
                rows_v.at[r], out_hbm.at[pl.ds(base, KCH)], ssem[r]).wait()

        fire_gathers(0, 0)
        fire_gathers(1, 1)

        @pl.loop(0, n_grp, step=2)
        def _(g):
            drain_gathers(0)
            fire_scatter(0, g)
            drain_gathers(1)
            fire_scatter(1, g + 1)
            drain_scatter(0)

            @pl.when(g + 2 < n_grp)
            def _():
                fire_gathers(0, g + 2)

            drain_scatter(1)

            @pl.when(g + 3 < n_grp)
            def _():
                fire_gathers(1, g + 3)

    return gather_kernel


def kernel(x, W):
    b, l = x.shape
    idx = x.reshape(-1)
    out = _make_gather(idx.shape[0], W.shape[0])(idx, W)
    return out.reshape(b, l, DIM)
